# baseline (speedup 1.0000x reference)
import jax
import jax.numpy as jnp
from jax import lax
from jax.experimental import pallas as pl
from jax.experimental.pallas import tpu as pltpu
from jax.experimental.pallas import tpu_sc as plsc

N = 100000
T = 8
D = 128

_NC = 2
_NS = 16
_NW = _NC * _NS

_CH = 320
_SUB = (128, 128, 64)
_NFULL = N // _CH
_TAIL = N - _NFULL * _CH
_SLOTS = -(-_NFULL // _NW)
_LAST_N = _NFULL - (_NW - 1) * _SLOTS


_B = 3
_SCALE = float(D) ** 0.5
_VPR = D // 16


def _gather_body(ids_hbm, tab_hbm, gam_hbm, bet_hbm, out_hbm, tab_sp, ids_v,
                 rows0, rows1, rows2, g0, g1, g2, s0, s1, s2):
    w = lax.axis_index("s") * _NC + lax.axis_index("c")
    c0 = w * _SLOTS
    n = jnp.minimum(_SLOTS, _NFULL - c0)
    row0 = c0 * _CH

    @pl.when(lax.axis_index("s") == 0)
    def _():
        pltpu.sync_copy(tab_hbm, rows0.at[pl.ds(0, T)])
        pltpu.sync_copy(gam_hbm, rows0.at[pl.ds(T, T)])
        pltpu.sync_copy(bet_hbm, rows0.at[pl.ds(2 * T, T)])
        def splat_sum(v):
            acc = v
            for sh in (8, 4, 2, 1):
                rows2[0, pl.ds(0, 16)] = acc
                rows2[0, pl.ds(16, 16)] = acc
                acc = acc + rows2[0, pl.ds(sh, 16)]
            return acc

        for t in range(T):
            xs = [rows0[t, pl.ds(16 * k, 16)] * _SCALE for k in range(_VPR)]
            tot = xs[0]
            for v in xs[1:]:
                tot = tot + v
            mean = splat_sum(tot) * (1.0 / D)
            xc = [x - mean for x in xs]
            sq = xc[0] * xc[0]
            for v in xc[1:]:
                sq = sq + v * v
            vv = splat_sum(sq) * (1.0 / D) + 1e-5
            i = lax.bitcast_convert_type(vv, jnp.int32)
            i = 0x5F3759DF - lax.shift_right_logical(i, 1)
            y = lax.bitcast_convert_type(i, jnp.float32)
            for _ in range(3):
                y = y * (1.5 - 0.5 * vv * y * y)
            for k in range(_VPR):
                g = rows0[T + t, pl.ds(16 * k, 16)]
                b = rows0[2 * T + t, pl.ds(16 * k, 16)]
                rows1[t, pl.ds(16 * k, 16)] = xc[k] * y * g + b
        pltpu.sync_copy(rows1.at[pl.ds(0, T)], tab_sp)

    plsc.subcore_barrier()

    @pl.when(w < _NW - 1)
    def _():
        pltpu.sync_copy(ids_hbm.at[pl.ds(row0, _SLOTS * _CH)], ids_v)

    @pl.when(w == _NW - 1)
    def _():
        cnt = _LAST_N * _CH + _TAIL
        pltpu.sync_copy(ids_hbm.at[pl.ds(row0, cnt)], ids_v.at[pl.ds(0, cnt)])

    bufs = (rows0, rows1, rows2)
    gsems = (g0, g1, g2)
    ssems = (s0, s1, s2)

    def gather_descs(j, buf, sem):
        ds, off = [], 0
        for ln in _SUB:
            idx = ids_v.at[pl.ds(j * _CH + off, ln)]
            ds.append(pltpu.make_async_copy(
                tab_sp.at[idx], buf.at[pl.ds(off, ln)], sem))
            off += ln
        return ds

    def store_desc(j, buf, sem):
        return pltpu.make_async_copy(
            buf, out_hbm.at[pl.ds((c0 + j) * _CH, _CH)], sem)

    def start_gather(j, buf, sem):
        for d in gather_descs(j, buf, sem):
            d.start()

    def wait_gather(j, buf, sem):
        for d in gather_descs(j, buf, sem):
            d.wait()

    start_gather(0, rows0, g0)
    start_gather(1, rows1, g1)

    @pl.when(n > 2)
    def _():
        start_gather(2, rows2, g2)

    def body(j, carry):
        @pl.when((j >= 1) & (j - 1 + _B < n))
        def _():
            for b in range(_B):
                @pl.when((j - 1) % _B == b)
                def _(b=b):
                    store_desc(j - 1, bufs[b], ssems[b]).wait()
                    start_gather(j - 1 + _B, bufs[b], gsems[b])

        for b in range(_B):
            @pl.when(j % _B == b)
            def _(b=b):
                wait_gather(j, bufs[b], gsems[b])
                store_desc(j, bufs[b], ssems[b]).start()

        return carry

    lax.fori_loop(0, n, body, 0)

    def drain(j, carry):
        for b in range(_B):
            @pl.when(j % _B == b)
            def _(b=b):
                store_desc(j, bufs[b], ssems[b]).wait()

        return carry

    lax.fori_loop(jnp.maximum(n - _B, 0), n, drain, 0)

    @pl.when(w == _NW - 1)
    def _():
        base = _LAST_N * _CH
        d1 = pltpu.make_async_copy(
            tab_sp.at[ids_v.at[pl.ds(base, 128)]], rows0.at[pl.ds(0, 128)], g0)
        d2 = pltpu.make_async_copy(
            tab_sp.at[ids_v.at[pl.ds(base + 128, _TAIL - 128)]],
            rows0.at[pl.ds(128, _TAIL - 128)], g0)
        d1.start()
        d2.start()
        d1.wait()
        d2.wait()
        pltpu.sync_copy(rows0.at[pl.ds(0, _TAIL)],
                        out_hbm.at[pl.ds(_NFULL * _CH, _TAIL)])


def kernel(node_type_ids, table, ln_gamma, ln_beta):
    mesh = plsc.VectorSubcoreMesh(core_axis_name="c", subcore_axis_name="s")
    gather = pl.kernel(
        _gather_body,
        mesh=mesh,
        out_type=jax.ShapeDtypeStruct((N, D), jnp.float32),
        scratch_types=[
            pltpu.VMEM_SHARED((T, D), jnp.float32),
            pltpu.VMEM((_SLOTS * _CH,), jnp.int32),
            pltpu.VMEM((_CH, D), jnp.float32),
            pltpu.VMEM((_CH, D), jnp.float32),
            pltpu.VMEM((_CH, D), jnp.float32),
            pltpu.SemaphoreType.DMA,
            pltpu.SemaphoreType.DMA,
            pltpu.SemaphoreType.DMA,
            pltpu.SemaphoreType.DMA,
            pltpu.SemaphoreType.DMA,
            pltpu.SemaphoreType.DMA,
        ],
    )
    return gather(node_type_ids.astype(jnp.int32), table, ln_gamma, ln_beta)


# speedup vs baseline: 1.0413x; 1.0413x over previous
import jax
import jax.numpy as jnp
from jax import lax
from jax.experimental import pallas as pl
from jax.experimental.pallas import tpu as pltpu
from jax.experimental.pallas import tpu_sc as plsc

N = 100000
T = 8
D = 128

_NC = 2
_NS = 16
_NW = _NC * _NS

_CH = 320
_SUB = (128, 128, 64)
_NFULL = N // _CH
_TAIL = N - _NFULL * _CH
_SLOTS = -(-_NFULL // _NW)
_LAST_N = _NFULL - (_NW - 1) * _SLOTS


_B = 3
_SCALE = float(D) ** 0.5
_VPR = D // 16


def _gather_body(ids_hbm, tab_hbm, gam_hbm, bet_hbm, out_hbm, tab_sp, ids_v,
                 rows0, rows1, rows2, g0, g1, g2, s0, s1, s2):
    w = lax.axis_index("s") * _NC + lax.axis_index("c")
    c0 = w * _SLOTS
    n = jnp.minimum(_SLOTS, _NFULL - c0)
    row0 = c0 * _CH

    @pl.when(w < _NW - 1)
    def _():
        pltpu.make_async_copy(
            ids_hbm.at[pl.ds(row0, _SLOTS * _CH)], ids_v, s0).start()

    @pl.when(w == _NW - 1)
    def _():
        cnt = _LAST_N * _CH + _TAIL
        pltpu.make_async_copy(
            ids_hbm.at[pl.ds(row0, cnt)], ids_v.at[pl.ds(0, cnt)], s0).start()

    @pl.when(lax.axis_index("s") == 0)
    def _():
        d1 = pltpu.make_async_copy(tab_hbm, rows0.at[pl.ds(0, T)], g0)
        d2 = pltpu.make_async_copy(gam_hbm, rows0.at[pl.ds(T, T)], g1)
        d3 = pltpu.make_async_copy(bet_hbm, rows0.at[pl.ds(2 * T, T)], g2)
        d1.start()
        d2.start()
        d3.start()
        d1.wait()
        d2.wait()
        d3.wait()
        def splat_sum(v):
            acc = v
            for sh in (8, 4, 2, 1):
                rows2[0, pl.ds(0, 16)] = acc
                rows2[0, pl.ds(16, 16)] = acc
                acc = acc + rows2[0, pl.ds(sh, 16)]
            return acc

        for t in range(T):
            xs = [rows0[t, pl.ds(16 * k, 16)] * _SCALE for k in range(_VPR)]
            tot = xs[0]
            for v in xs[1:]:
                tot = tot + v
            mean = splat_sum(tot) * (1.0 / D)
            xc = [x - mean for x in xs]
            sq = xc[0] * xc[0]
            for v in xc[1:]:
                sq = sq + v * v
            vv = splat_sum(sq) * (1.0 / D) + 1e-5
            i = lax.bitcast_convert_type(vv, jnp.int32)
            i = 0x5F3759DF - lax.shift_right_logical(i, 1)
            y = lax.bitcast_convert_type(i, jnp.float32)
            for _ in range(3):
                y = y * (1.5 - 0.5 * vv * y * y)
            for k in range(_VPR):
                g = rows0[T + t, pl.ds(16 * k, 16)]
                b = rows0[2 * T + t, pl.ds(16 * k, 16)]
                rows1[t, pl.ds(16 * k, 16)] = xc[k] * y * g + b
        pltpu.sync_copy(rows1.at[pl.ds(0, T)], tab_sp)

    plsc.subcore_barrier()

    @pl.when(w < _NW - 1)
    def _():
        pltpu.make_async_copy(
            ids_hbm.at[pl.ds(row0, _SLOTS * _CH)], ids_v, s0).wait()

    @pl.when(w == _NW - 1)
    def _():
        cnt = _LAST_N * _CH + _TAIL
        pltpu.make_async_copy(
            ids_hbm.at[pl.ds(row0, cnt)], ids_v.at[pl.ds(0, cnt)], s0).wait()

    bufs = (rows0, rows1, rows2)
    gsems = (g0, g1, g2)
    ssems = (s0, s1, s2)

    def gather_descs(j, buf, sem):
        ds, off = [], 0
        for ln in _SUB:
            idx = ids_v.at[pl.ds(j * _CH + off, ln)]
            ds.append(pltpu.make_async_copy(
                tab_sp.at[idx], buf.at[pl.ds(off, ln)], sem))
            off += ln
        return ds

    def store_desc(j, buf, sem):
        return pltpu.make_async_copy(
            buf, out_hbm.at[pl.ds((c0 + j) * _CH, _CH)], sem)

    def start_gather(j, buf, sem):
        for d in gather_descs(j, buf, sem):
            d.start()

    def wait_gather(j, buf, sem):
        for d in gather_descs(j, buf, sem):
            d.wait()

    start_gather(0, rows0, g0)
    start_gather(1, rows1, g1)

    @pl.when(n > 2)
    def _():
        start_gather(2, rows2, g2)

    def body(j, carry):
        @pl.when((j >= 1) & (j - 1 + _B < n))
        def _():
            for b in range(_B):
                @pl.when((j - 1) % _B == b)
                def _(b=b):
                    store_desc(j - 1, bufs[b], ssems[b]).wait()
                    start_gather(j - 1 + _B, bufs[b], gsems[b])

        for b in range(_B):
            @pl.when(j % _B == b)
            def _(b=b):
                wait_gather(j, bufs[b], gsems[b])
                store_desc(j, bufs[b], ssems[b]).start()

        return carry

    lax.fori_loop(0, n, body, 0)

    def drain(j, carry):
        for b in range(_B):
            @pl.when(j % _B == b)
            def _(b=b):
                store_desc(j, bufs[b], ssems[b]).wait()

        return carry

    lax.fori_loop(jnp.maximum(n - _B, 0), n, drain, 0)

    @pl.when(w == _NW - 1)
    def _():
        base = _LAST_N * _CH
        d1 = pltpu.make_async_copy(
            tab_sp.at[ids_v.at[pl.ds(base, 128)]], rows0.at[pl.ds(0, 128)], g0)
        d2 = pltpu.make_async_copy(
            tab_sp.at[ids_v.at[pl.ds(base + 128, _TAIL - 128)]],
            rows0.at[pl.ds(128, _TAIL - 128)], g0)
        d1.start()
        d2.start()
        d1.wait()
        d2.wait()
        pltpu.sync_copy(rows0.at[pl.ds(0, _TAIL)],
                        out_hbm.at[pl.ds(_NFULL * _CH, _TAIL)])


def kernel(node_type_ids, table, ln_gamma, ln_beta):
    mesh = plsc.VectorSubcoreMesh(core_axis_name="c", subcore_axis_name="s")
    gather = pl.kernel(
        _gather_body,
        mesh=mesh,
        out_type=jax.ShapeDtypeStruct((N, D), jnp.float32),
        scratch_types=[
            pltpu.VMEM_SHARED((T, D), jnp.float32),
            pltpu.VMEM((_SLOTS * _CH,), jnp.int32),
            pltpu.VMEM((_CH, D), jnp.float32),
            pltpu.VMEM((_CH, D), jnp.float32),
            pltpu.VMEM((_CH, D), jnp.float32),
            pltpu.SemaphoreType.DMA,
            pltpu.SemaphoreType.DMA,
            pltpu.SemaphoreType.DMA,
            pltpu.SemaphoreType.DMA,
            pltpu.SemaphoreType.DMA,
            pltpu.SemaphoreType.DMA,
        ],
    )
    return gather(node_type_ids.astype(jnp.int32), table, ln_gamma, ln_beta)


# speedup vs baseline: 1.0891x; 1.0458x over previous
import jax
import jax.numpy as jnp
from jax import lax
from jax.experimental import pallas as pl
from jax.experimental.pallas import tpu as pltpu
from jax.experimental.pallas import tpu_sc as plsc

N = 100000
T = 8
D = 128

_NC = 2
_NS = 16
_NW = _NC * _NS

_CH = 320
_SUB = (128, 128, 64)
_NFULL = N // _CH
_TAIL = N - _NFULL * _CH
_SLOTS = -(-_NFULL // _NW)
_LAST_N = _NFULL - (_NW - 1) * _SLOTS


def _ln_table_kernel(table_ref, gamma_ref, beta_ref, out_ref):
    x = table_ref[...] * (D ** 0.5)
    mean = jnp.mean(x, axis=-1, keepdims=True)
    xc = x - mean
    var = jnp.mean(xc * xc, axis=-1, keepdims=True)
    out_ref[...] = xc * lax.rsqrt(var + 1e-5) * gamma_ref[...] + beta_ref[...]


def _normed_table(table, ln_gamma, ln_beta):
    return pl.pallas_call(
        _ln_table_kernel,
        out_shape=jax.ShapeDtypeStruct((T, D), jnp.float32),
    )(table, ln_gamma, ln_beta)


_B = 3
_SCALE = float(D) ** 0.5
_VPR = D // 16


def _gather_body(ids_hbm, tab_hbm, out_hbm, tab_sp, ids_v,
                 rows0, rows1, rows2, g0, g1, g2, s0, s1, s2):
    w = lax.axis_index("s") * _NC + lax.axis_index("c")
    c0 = w * _SLOTS
    n = jnp.minimum(_SLOTS, _NFULL - c0)
    row0 = c0 * _CH

    @pl.when(w < _NW - 1)
    def _():
        pltpu.make_async_copy(
            ids_hbm.at[pl.ds(row0, _SLOTS * _CH)], ids_v, s0).start()

    @pl.when(w == _NW - 1)
    def _():
        cnt = _LAST_N * _CH + _TAIL
        pltpu.make_async_copy(
            ids_hbm.at[pl.ds(row0, cnt)], ids_v.at[pl.ds(0, cnt)], s0).start()

    @pl.when(lax.axis_index("s") == 0)
    def _():
        pltpu.sync_copy(tab_hbm, tab_sp)

    plsc.subcore_barrier()

    @pl.when(w < _NW - 1)
    def _():
        pltpu.make_async_copy(
            ids_hbm.at[pl.ds(row0, _SLOTS * _CH)], ids_v, s0).wait()

    @pl.when(w == _NW - 1)
    def _():
        cnt = _LAST_N * _CH + _TAIL
        pltpu.make_async_copy(
            ids_hbm.at[pl.ds(row0, cnt)], ids_v.at[pl.ds(0, cnt)], s0).wait()

    bufs = (rows0, rows1, rows2)
    gsems = (g0, g1, g2)
    ssems = (s0, s1, s2)

    def gather_descs(j, buf, sem):
        ds, off = [], 0
        for ln in _SUB:
            idx = ids_v.at[pl.ds(j * _CH + off, ln)]
            ds.append(pltpu.make_async_copy(
                tab_sp.at[idx], buf.at[pl.ds(off, ln)], sem))
            off += ln
        return ds

    def store_desc(j, buf, sem):
        return pltpu.make_async_copy(
            buf, out_hbm.at[pl.ds((c0 + j) * _CH, _CH)], sem)

    def start_gather(j, buf, sem):
        for d in gather_descs(j, buf, sem):
            d.start()

    def wait_gather(j, buf, sem):
        for d in gather_descs(j, buf, sem):
            d.wait()

    start_gather(0, rows0, g0)
    start_gather(1, rows1, g1)

    @pl.when(n > 2)
    def _():
        start_gather(2, rows2, g2)

    def body(j, carry):
        @pl.when((j >= 1) & (j - 1 + _B < n))
        def _():
            for b in range(_B):
                @pl.when((j - 1) % _B == b)
                def _(b=b):
                    store_desc(j - 1, bufs[b], ssems[b]).wait()
                    start_gather(j - 1 + _B, bufs[b], gsems[b])

        for b in range(_B):
            @pl.when(j % _B == b)
            def _(b=b):
                wait_gather(j, bufs[b], gsems[b])
                store_desc(j, bufs[b], ssems[b]).start()

        return carry

    lax.fori_loop(0, n, body, 0)

    def drain(j, carry):
        for b in range(_B):
            @pl.when(j % _B == b)
            def _(b=b):
                store_desc(j, bufs[b], ssems[b]).wait()

        return carry

    lax.fori_loop(jnp.maximum(n - _B, 0), n, drain, 0)

    @pl.when(w == _NW - 1)
    def _():
        base = _LAST_N * _CH
        d1 = pltpu.make_async_copy(
            tab_sp.at[ids_v.at[pl.ds(base, 128)]], rows0.at[pl.ds(0, 128)], g0)
        d2 = pltpu.make_async_copy(
            tab_sp.at[ids_v.at[pl.ds(base + 128, _TAIL - 128)]],
            rows0.at[pl.ds(128, _TAIL - 128)], g0)
        d1.start()
        d2.start()
        d1.wait()
        d2.wait()
        pltpu.sync_copy(rows0.at[pl.ds(0, _TAIL)],
                        out_hbm.at[pl.ds(_NFULL * _CH, _TAIL)])


def kernel(node_type_ids, table, ln_gamma, ln_beta):
    normed = _normed_table(table, ln_gamma, ln_beta)
    mesh = plsc.VectorSubcoreMesh(core_axis_name="c", subcore_axis_name="s")
    gather = pl.kernel(
        _gather_body,
        mesh=mesh,
        out_type=jax.ShapeDtypeStruct((N, D), jnp.float32),
        scratch_types=[
            pltpu.VMEM_SHARED((T, D), jnp.float32),
            pltpu.VMEM((_SLOTS * _CH,), jnp.int32),
            pltpu.VMEM((_CH, D), jnp.float32),
            pltpu.VMEM((_CH, D), jnp.float32),
            pltpu.VMEM((_CH, D), jnp.float32),
            pltpu.SemaphoreType.DMA,
            pltpu.SemaphoreType.DMA,
            pltpu.SemaphoreType.DMA,
            pltpu.SemaphoreType.DMA,
            pltpu.SemaphoreType.DMA,
            pltpu.SemaphoreType.DMA,
        ],
    )
    return gather(node_type_ids.astype(jnp.int32), normed)


# speedup vs baseline: 1.0935x; 1.0041x over previous
import jax
import jax.numpy as jnp
from jax import lax
from jax.experimental import pallas as pl
from jax.experimental.pallas import tpu as pltpu
from jax.experimental.pallas import tpu_sc as plsc

N = 100000
T = 8
D = 128

_NC = 2
_NS = 16
_NW = _NC * _NS

_CH = 320
_SUB = (320,)
_NFULL = N // _CH
_TAIL = N - _NFULL * _CH
_SLOTS = -(-_NFULL // _NW)
_LAST_N = _NFULL - (_NW - 1) * _SLOTS


def _ln_table_kernel(table_ref, gamma_ref, beta_ref, out_ref):
    x = table_ref[...] * (D ** 0.5)
    mean = jnp.mean(x, axis=-1, keepdims=True)
    xc = x - mean
    var = jnp.mean(xc * xc, axis=-1, keepdims=True)
    out_ref[...] = xc * lax.rsqrt(var + 1e-5) * gamma_ref[...] + beta_ref[...]


def _normed_table(table, ln_gamma, ln_beta):
    return pl.pallas_call(
        _ln_table_kernel,
        out_shape=jax.ShapeDtypeStruct((T, D), jnp.float32),
    )(table, ln_gamma, ln_beta)


_B = 3


def _gather_body(ids_hbm, tab_hbm, out_hbm, tab_sp, ids_v,
                 rows0, rows1, rows2, g0, g1, g2, s0, s1, s2):
    w = lax.axis_index("s") * _NC + lax.axis_index("c")
    c0 = w * _SLOTS
    n = jnp.minimum(_SLOTS, _NFULL - c0)
    row0 = c0 * _CH

    @pl.when(w < _NW - 1)
    def _():
        pltpu.make_async_copy(
            ids_hbm.at[pl.ds(row0, _SLOTS * _CH)], ids_v, s0).start()

    @pl.when(w == _NW - 1)
    def _():
        cnt = _LAST_N * _CH + _TAIL
        pltpu.make_async_copy(
            ids_hbm.at[pl.ds(row0, cnt)], ids_v.at[pl.ds(0, cnt)], s0).start()

    @pl.when(lax.axis_index("s") == 0)
    def _():
        pltpu.sync_copy(tab_hbm, tab_sp)

    plsc.subcore_barrier()

    @pl.when(w < _NW - 1)
    def _():
        pltpu.make_async_copy(
            ids_hbm.at[pl.ds(row0, _SLOTS * _CH)], ids_v, s0).wait()

    @pl.when(w == _NW - 1)
    def _():
        cnt = _LAST_N * _CH + _TAIL
        pltpu.make_async_copy(
            ids_hbm.at[pl.ds(row0, cnt)], ids_v.at[pl.ds(0, cnt)], s0).wait()

    bufs = (rows0, rows1, rows2)
    gsems = (g0, g1, g2)
    ssems = (s0, s1, s2)

    def gather_descs(j, buf, sem):
        ds, off = [], 0
        for ln in _SUB:
            idx = ids_v.at[pl.ds(j * _CH + off, ln)]
            ds.append(pltpu.make_async_copy(
                tab_sp.at[idx], buf.at[pl.ds(off, ln)], sem))
            off += ln
        return ds

    def store_desc(j, buf, sem):
        return pltpu.make_async_copy(
            buf, out_hbm.at[pl.ds((c0 + j) * _CH, _CH)], sem)

    def start_gather(j, buf, sem):
        for d in gather_descs(j, buf, sem):
            d.start()

    def wait_gather(j, buf, sem):
        for d in gather_descs(j, buf, sem):
            d.wait()

    start_gather(0, rows0, g0)
    start_gather(1, rows1, g1)

    @pl.when(n > 2)
    def _():
        start_gather(2, rows2, g2)

    def body(j, carry):
        @pl.when((j >= 1) & (j - 1 + _B < n))
        def _():
            for b in range(_B):
                @pl.when((j - 1) % _B == b)
                def _(b=b):
                    store_desc(j - 1, bufs[b], ssems[b]).wait()
                    start_gather(j - 1 + _B, bufs[b], gsems[b])

        for b in range(_B):
            @pl.when(j % _B == b)
            def _(b=b):
                wait_gather(j, bufs[b], gsems[b])
                store_desc(j, bufs[b], ssems[b]).start()

        return carry

    lax.fori_loop(0, n, body, 0)

    def drain(j, carry):
        for b in range(_B):
            @pl.when(j % _B == b)
            def _(b=b):
                store_desc(j, bufs[b], ssems[b]).wait()

        return carry

    lax.fori_loop(jnp.maximum(n - _B, 0), n, drain, 0)

    @pl.when(w == _NW - 1)
    def _():
        base = _LAST_N * _CH
        d1 = pltpu.make_async_copy(
            tab_sp.at[ids_v.at[pl.ds(base, 128)]], rows0.at[pl.ds(0, 128)], g0)
        d2 = pltpu.make_async_copy(
            tab_sp.at[ids_v.at[pl.ds(base + 128, _TAIL - 128)]],
            rows0.at[pl.ds(128, _TAIL - 128)], g0)
        d1.start()
        d2.start()
        d1.wait()
        d2.wait()
        pltpu.sync_copy(rows0.at[pl.ds(0, _TAIL)],
                        out_hbm.at[pl.ds(_NFULL * _CH, _TAIL)])


def kernel(node_type_ids, table, ln_gamma, ln_beta):
    normed = _normed_table(table, ln_gamma, ln_beta)
    mesh = plsc.VectorSubcoreMesh(core_axis_name="c", subcore_axis_name="s")
    gather = pl.kernel(
        _gather_body,
        mesh=mesh,
        out_type=jax.ShapeDtypeStruct((N, D), jnp.float32),
        scratch_types=[
            pltpu.VMEM_SHARED((T, D), jnp.float32),
            pltpu.VMEM((_SLOTS * _CH,), jnp.int32),
            pltpu.VMEM((_CH, D), jnp.float32),
            pltpu.VMEM((_CH, D), jnp.float32),
            pltpu.VMEM((_CH, D), jnp.float32),
            pltpu.SemaphoreType.DMA,
            pltpu.SemaphoreType.DMA,
            pltpu.SemaphoreType.DMA,
            pltpu.SemaphoreType.DMA,
            pltpu.SemaphoreType.DMA,
            pltpu.SemaphoreType.DMA,
        ],
    )
    return gather(node_type_ids.astype(jnp.int32), normed)
